# Initial kernel scaffold; baseline (speedup 1.0000x reference)
import jax
import jax.numpy as jnp
from jax.experimental import pallas as pl


def kernel(x, edge_index, edge_attr, W1, b1, W2, b2, W3, b3):
    raise NotImplementedError("write your pallas kernel here")


# speedup vs baseline: 4.5419x; 4.5419x over previous
import functools

import jax
import jax.numpy as jnp
from jax import lax
from jax.experimental import pallas as pl
from jax.experimental.pallas import tpu as pltpu
from jax.experimental.pallas import tpu_sc as plsc

N = 10000
NP = 10240
E = 320000
NC = 2
NS = 16
F32 = jnp.float32


def _sc_mesh():
    return plsc.VectorSubcoreMesh(core_axis_name="c", subcore_axis_name="s")


@functools.cache
def _make_prop(feat_split):
    B = 80
    W = 128
    NT = NS if feat_split else NC * NS
    EPT = E // NT
    NCH = EPT // B
    RPT = NP // NS

    @functools.partial(
        pl.kernel,
        out_type=jax.ShapeDtypeStruct((NC, NP, W), F32),
        mesh=_sc_mesh(),
        scratch_types=[
            pltpu.VMEM((B,), jnp.int32),
            pltpu.VMEM((B,), jnp.int32),
            pltpu.VMEM((B,), jnp.int32),
            pltpu.VMEM((B, W), F32),
            pltpu.VMEM_SHARED((NP, W), F32),
            pltpu.SemaphoreType.DMA,
        ],
    )
    def prop_kernel(g2_hbm, src_hbm, dst_hbm, zeros_hbm, out_hbm,
                    src_v, dst_v, idx_v, rows_v, acc, sem):
        c = lax.axis_index("c")
        s = lax.axis_index("s")

        pltpu.sync_copy(zeros_hbm.at[pl.ds(s * RPT, RPT)],
                        acc.at[pl.ds(s * RPT, RPT)])
        plsc.subcore_barrier()

        e0 = (s if feat_split else s * NC + c) * EPT

        def chunk(k, _):
            base = e0 + k * B
            pltpu.sync_copy(src_hbm.at[pl.ds(base, B)], src_v)
            pltpu.sync_copy(dst_hbm.at[pl.ds(base, B)], dst_v)
            if feat_split:
                for j in range(B // 16):
                    sv = src_v[pl.ds(j * 16, 16)]
                    idx_v[pl.ds(j * 16, 16)] = sv * 2 + c
                gidx = idx_v
            else:
                gidx = src_v
            pltpu.async_copy(g2_hbm.at[gidx], rows_v, sem).wait()
            pltpu.sync_copy(rows_v, acc.at[dst_v], add=True)
            return 0
        lax.fori_loop(0, NCH, chunk, 0)
        plsc.subcore_barrier()
        pltpu.sync_copy(acc.at[pl.ds(s * RPT, RPT)],
                        out_hbm.at[c, pl.ds(s * RPT, RPT)])

    return prop_kernel


_R = 2000


def _tc_call(body, out_shapes, in_specs, out_specs):
    return pl.pallas_call(
        body,
        grid=(N // _R,),
        in_specs=in_specs,
        out_specs=out_specs,
        out_shape=out_shapes,
    )


def _rows(c):
    return pl.BlockSpec((_R, c), lambda i: (i, 0))


def _pair(w):
    return pl.BlockSpec((NC, _R, w), lambda i: (0, i, 0))


def _full(a, b):
    return pl.BlockSpec((a, b), lambda i: (0, 0))


def _dinv_g0(deg_p, x):
    def body(degp_ref, x_ref, dinvb_ref, g0_ref):
        deg = degp_ref[0][:, 0:1] + degp_ref[1][:, 0:1]
        dinv = jnp.where(deg > 0.0, lax.rsqrt(deg), 0.0)
        dinvb_ref[...] = jnp.broadcast_to(dinv, (_R, 128))
        g0_ref[...] = dinv * x_ref[...]

    return _tc_call(
        body,
        (jax.ShapeDtypeStruct((N, 128), F32), jax.ShapeDtypeStruct((N, 128), F32)),
        [_pair(128), _rows(128)],
        (_rows(128), _rows(128)),
    )(deg_p, x)


def _mid(h, s1p, dinvb, w0, w1):
    ci, co = w0.shape

    def body(h_ref, sp_ref, dv_ref, w0_ref, w1_ref, out01_ref, g1_ref):
        dv = dv_ref[:, 0:1]
        if ci == 256:
            s1 = jnp.concatenate([sp_ref[0], sp_ref[1]], axis=-1)
        else:
            s1 = sp_ref[0] + sp_ref[1]
        tx1 = (-dv) * s1
        out01_ref[...] = (
            jnp.dot(h_ref[...], w0_ref[...], preferred_element_type=F32)
            + jnp.dot(tx1, w1_ref[...], preferred_element_type=F32))
        g1_ref[...] = dv * tx1

    return _tc_call(
        body,
        (jax.ShapeDtypeStruct((N, co), F32), jax.ShapeDtypeStruct((N, ci), F32)),
        [_rows(ci), _pair(128), _rows(128), _full(ci, co), _full(ci, co)],
        (_rows(co), _rows(ci)),
    )(h, s1p, dinvb, w0, w1)


def _fin(out01, s2p, h, dinvb, w2, b, last):
    ci, co = w2.shape

    def body(o_ref, sp_ref, h_ref, dv_ref, w2_ref, b_ref, *outs):
        dv = dv_ref[:, 0:1]
        if ci == 256:
            s2 = jnp.concatenate([sp_ref[0], sp_ref[1]], axis=-1)
        else:
            s2 = sp_ref[0] + sp_ref[1]
        tx2 = (-2.0 * dv) * s2 - h_ref[...]
        out = (o_ref[...] + b_ref[...]
               + jnp.dot(tx2, w2_ref[...], preferred_element_type=F32))
        a = jnp.maximum(out, 0.0)
        if last:
            m = jnp.max(a, axis=-1, keepdims=True)
            lse = m + jnp.log(jnp.sum(jnp.exp(a - m), axis=-1, keepdims=True))
            outs[0][...] = a - lse
        else:
            outs[0][...] = a
            outs[1][...] = dv * a

    n_out = 1 if last else 2
    return _tc_call(
        body,
        tuple(jax.ShapeDtypeStruct((N, co), F32) for _ in range(n_out)),
        [_rows(co), _pair(128), _rows(ci), _rows(128), _full(ci, co),
         pl.BlockSpec((1, co), lambda i: (0, 0))],
        tuple(_rows(co) for _ in range(n_out)),
    )(out01, s2p, h, dinvb, w2, b.reshape(1, co))


@jax.jit
def kernel(x, edge_index, edge_attr, W1, b1, W2, b2, W3, b3):
    src = edge_index[0]
    dst = edge_index[1]
    zeros128 = jnp.zeros((NP, 128), F32)

    ones128 = jnp.ones((N, 128), F32)
    deg_p = _make_prop(False)(ones128, dst, src, zeros128)
    dinvb, g0 = _dinv_g0(deg_p, x)

    h = x
    g = g0
    for li, (W, b) in enumerate(((W1, b1), (W2, b2), (W3, b3))):
        ci = W.shape[1]
        feat = ci == 256
        prop = _make_prop(feat)
        s1p = prop(g.reshape(-1, 128), src, dst, zeros128)
        out01, g1 = _mid(h, s1p, dinvb, W[0], W[1])
        s2p = prop(g1.reshape(-1, 128), src, dst, zeros128)
        last = li == 2
        res = _fin(out01, s2p, h, dinvb, W[2], b, last)
        if last:
            return res[0]
        h, g = res


# speedup vs baseline: 12.1738x; 2.6804x over previous
import functools

import jax
import jax.numpy as jnp
from jax import lax
from jax.experimental import pallas as pl
from jax.experimental.pallas import tpu as pltpu
from jax.experimental.pallas import tpu_sc as plsc

N = 10000
NP = 10240
E = 320000
NC = 2
NS = 16
F32 = jnp.float32


def _sc_mesh():
    return plsc.VectorSubcoreMesh(core_axis_name="c", subcore_axis_name="s")


@functools.cache
def _make_prop(feat_split):
    B = 80
    CPG = 2
    EPG = B * CPG
    W = 128
    NT = NS if feat_split else NC * NS
    EPT = E // NT
    G = EPT // EPG
    TAIL = (EPT % EPG) // B
    RPT = NP // NS

    scratch = (
        [pltpu.VMEM((EPG,), jnp.int32)] * 4 +
        [pltpu.VMEM((B,), jnp.int32)] * (4 * CPG) +
        [pltpu.VMEM((B, W), F32)] * (2 * CPG) +
        [pltpu.VMEM_SHARED((NP, W), F32)] +
        [pltpu.SemaphoreType.DMA] * 4
    )

    @functools.partial(
        pl.kernel,
        out_type=jax.ShapeDtypeStruct((NC, NP, W), F32),
        mesh=_sc_mesh(),
        scratch_types=scratch,
    )
    def prop_kernel(g2_hbm, src_hbm, dst_hbm, zeros_hbm, out_hbm, *sc):
        sblk = sc[0:2]
        dblk = sc[2:4]
        gidx = (sc[4:4 + CPG], sc[4 + CPG:4 + 2 * CPG])
        didx = (sc[4 + 2 * CPG:4 + 3 * CPG], sc[4 + 3 * CPG:4 + 4 * CPG])
        o = 4 + 4 * CPG
        rows = (sc[o:o + CPG], sc[o + CPG:o + 2 * CPG])
        acc = sc[o + 2 * CPG]
        gsem = sc[o + 2 * CPG + 1:o + 2 * CPG + 3]
        ssem = sc[o + 2 * CPG + 3:o + 2 * CPG + 5]

        c = lax.axis_index("c")
        s = lax.axis_index("s")
        e0 = (s if feat_split else s * NC + c) * EPT

        pltpu.sync_copy(zeros_hbm.at[pl.ds(s * RPT, RPT)],
                        acc.at[pl.ds(s * RPT, RPT)])
        plsc.subcore_barrier()

        def load_blk(bk, g, n=EPG):
            base = e0 + g * EPG
            pltpu.sync_copy(src_hbm.at[pl.ds(base, n)], sblk[bk])
            pltpu.sync_copy(dst_hbm.at[pl.ds(base, n)], dblk[bk])

        def fire_gathers(bk, nch=CPG):
            for j in range(nch):
                for i in range(B // 16):
                    off = j * B + i * 16
                    s16 = sblk[bk][pl.ds(off, 16)]
                    if feat_split:
                        s16 = s16 * 2 + c
                    gidx[bk][j][pl.ds(i * 16, 16)] = s16
                    didx[bk][j][pl.ds(i * 16, 16)] = dblk[bk][pl.ds(off, 16)]
                pltpu.async_copy(g2_hbm.at[gidx[bk][j]], rows[bk][j], gsem[bk])

        def drain_gathers(bk, nch=CPG):
            for j in range(nch):
                pltpu.make_async_copy(
                    g2_hbm.at[gidx[bk][j]], rows[bk][j], gsem[bk]).wait()

        def fire_scatters(bk, nch=CPG):
            for j in range(nch):
                pltpu.async_copy(rows[bk][j], acc.at[didx[bk][j]], ssem[bk],
                                 add=True)

        def drain_scatters(bk, nch=CPG):
            for j in range(nch):
                pltpu.make_async_copy(
                    rows[bk][j], acc.at[didx[bk][j]], ssem[bk]).wait()

        load_blk(0, 0)
        fire_gathers(0)
        load_blk(1, 1)
        fire_gathers(1)

        def body(t, _):
            for bk in range(2):
                drain_gathers(bk)
                fire_scatters(bk)
                load_blk(bk, 2 * t + 2 + bk)
                drain_scatters(bk)
                fire_gathers(bk)
            return 0
        lax.fori_loop(0, (G - 2) // 2, body, 0)

        for bk in range(2):
            drain_gathers(bk)
            fire_scatters(bk)
            drain_scatters(bk)
        if G % 2:
            load_blk(0, G - 1)
            fire_gathers(0)
            drain_gathers(0)
            fire_scatters(0)
            drain_scatters(0)
        if TAIL:
            base = e0 + G * EPG
            pltpu.sync_copy(src_hbm.at[pl.ds(base, TAIL * B)],
                            sblk[0].at[pl.ds(0, TAIL * B)])
            pltpu.sync_copy(dst_hbm.at[pl.ds(base, TAIL * B)],
                            dblk[0].at[pl.ds(0, TAIL * B)])
            fire_gathers(0, TAIL)
            drain_gathers(0, TAIL)
            fire_scatters(0, TAIL)
            drain_scatters(0, TAIL)

        plsc.subcore_barrier()
        pltpu.sync_copy(acc.at[pl.ds(s * RPT, RPT)],
                        out_hbm.at[c, pl.ds(s * RPT, RPT)])

    return prop_kernel


@functools.cache
def _make_deg():
    B = 80
    CPG = 5
    EPG = B * CPG
    W = 128
    EPT = E // (NC * NS)
    G = EPT // EPG
    RPT = NP // NS

    scratch = (
        [pltpu.VMEM((EPG,), jnp.int32)] * 2 +
        [pltpu.VMEM((B,), jnp.int32)] * 10 +
        [pltpu.VMEM((B, W), F32)] +
        [pltpu.VMEM_SHARED((NP, W), F32)] +
        [pltpu.SemaphoreType.DMA] * 2
    )

    @functools.partial(
        pl.kernel,
        out_type=jax.ShapeDtypeStruct((NC, NP, W), F32),
        mesh=_sc_mesh(),
        scratch_types=scratch,
    )
    def deg_kernel(src_hbm, ones_hbm, zeros_hbm, out_hbm, *sc):
        sblk = sc[0:2]
        didx = (sc[2:7], sc[7:12])
        ones_v = sc[12]
        acc = sc[13]
        ssem = sc[14:16]

        c = lax.axis_index("c")
        s = lax.axis_index("s")
        e0 = (s * NC + c) * EPT

        pltpu.sync_copy(ones_hbm, ones_v)
        pltpu.sync_copy(zeros_hbm.at[pl.ds(s * RPT, RPT)],
                        acc.at[pl.ds(s * RPT, RPT)])
        plsc.subcore_barrier()

        def load_blk(bk, g):
            pltpu.sync_copy(src_hbm.at[pl.ds(e0 + g * EPG, EPG)], sblk[bk])

        def prep_fire(bk):
            for j in range(CPG):
                for i in range(B // 16):
                    didx[bk][j][pl.ds(i * 16, 16)] = \
                        sblk[bk][pl.ds(j * B + i * 16, 16)]
                pltpu.async_copy(ones_v, acc.at[didx[bk][j]], ssem[bk],
                                 add=True)

        def drain(bk):
            for j in range(CPG):
                pltpu.make_async_copy(
                    ones_v, acc.at[didx[bk][j]], ssem[bk]).wait()

        load_blk(0, 0)
        prep_fire(0)
        load_blk(1, 1)
        prep_fire(1)

        def body(t, _):
            for bk in range(2):
                drain(bk)
                load_blk(bk, 2 * t + 2 + bk)
                prep_fire(bk)
            return 0
        lax.fori_loop(0, (G - 2) // 2, body, 0)
        for bk in range(2):
            drain(bk)
        if G % 2:
            load_blk(0, G - 1)
            prep_fire(0)
            drain(0)

        plsc.subcore_barrier()
        pltpu.sync_copy(acc.at[pl.ds(s * RPT, RPT)],
                        out_hbm.at[c, pl.ds(s * RPT, RPT)])

    return deg_kernel


_R = 2000


def _tc_call(body, out_shapes, in_specs, out_specs):
    return pl.pallas_call(
        body,
        grid=(N // _R,),
        in_specs=in_specs,
        out_specs=out_specs,
        out_shape=out_shapes,
    )


def _rows(c):
    return pl.BlockSpec((_R, c), lambda i: (i, 0))


def _pair(w):
    return pl.BlockSpec((NC, _R, w), lambda i: (0, i, 0))


def _full(a, b):
    return pl.BlockSpec((a, b), lambda i: (0, 0))


def _dinv_g0(deg_p, x):
    def body(degp_ref, x_ref, dinvb_ref, g0_ref):
        deg = degp_ref[0][:, 0:1] + degp_ref[1][:, 0:1]
        dinv = jnp.where(deg > 0.0, lax.rsqrt(deg), 0.0)
        dinvb_ref[...] = jnp.broadcast_to(dinv, (_R, 128))
        g0_ref[...] = dinv * x_ref[...]

    return _tc_call(
        body,
        (jax.ShapeDtypeStruct((N, 128), F32), jax.ShapeDtypeStruct((N, 128), F32)),
        [_pair(128), _rows(128)],
        (_rows(128), _rows(128)),
    )(deg_p, x)


def _mid(h, s1p, dinvb, w0, w1):
    ci, co = w0.shape

    def body(h_ref, sp_ref, dv_ref, w0_ref, w1_ref, out01_ref, g1_ref):
        dv = dv_ref[:, 0:1]
        if ci == 256:
            s1 = jnp.concatenate([sp_ref[0], sp_ref[1]], axis=-1)
        else:
            s1 = sp_ref[0] + sp_ref[1]
        tx1 = (-dv) * s1
        out01_ref[...] = (
            jnp.dot(h_ref[...], w0_ref[...], preferred_element_type=F32)
            + jnp.dot(tx1, w1_ref[...], preferred_element_type=F32))
        g1_ref[...] = dv * tx1

    return _tc_call(
        body,
        (jax.ShapeDtypeStruct((N, co), F32), jax.ShapeDtypeStruct((N, ci), F32)),
        [_rows(ci), _pair(128), _rows(128), _full(ci, co), _full(ci, co)],
        (_rows(co), _rows(ci)),
    )(h, s1p, dinvb, w0, w1)


def _fin(out01, s2p, h, dinvb, w2, b, last):
    ci, co = w2.shape

    def body(o_ref, sp_ref, h_ref, dv_ref, w2_ref, b_ref, *outs):
        dv = dv_ref[:, 0:1]
        if ci == 256:
            s2 = jnp.concatenate([sp_ref[0], sp_ref[1]], axis=-1)
        else:
            s2 = sp_ref[0] + sp_ref[1]
        tx2 = (-2.0 * dv) * s2 - h_ref[...]
        out = (o_ref[...] + b_ref[...]
               + jnp.dot(tx2, w2_ref[...], preferred_element_type=F32))
        a = jnp.maximum(out, 0.0)
        if last:
            m = jnp.max(a, axis=-1, keepdims=True)
            lse = m + jnp.log(jnp.sum(jnp.exp(a - m), axis=-1, keepdims=True))
            outs[0][...] = a - lse
        else:
            outs[0][...] = a
            outs[1][...] = dv * a

    n_out = 1 if last else 2
    return _tc_call(
        body,
        tuple(jax.ShapeDtypeStruct((N, co), F32) for _ in range(n_out)),
        [_rows(co), _pair(128), _rows(ci), _rows(128), _full(ci, co),
         pl.BlockSpec((1, co), lambda i: (0, 0))],
        tuple(_rows(co) for _ in range(n_out)),
    )(out01, s2p, h, dinvb, w2, b.reshape(1, co))


@jax.jit
def kernel(x, edge_index, edge_attr, W1, b1, W2, b2, W3, b3):
    src = edge_index[0]
    dst = edge_index[1]
    zeros128 = jnp.zeros((NP, 128), F32)

    ones80 = jnp.ones((80, 128), F32)
    deg_p = _make_deg()(src, ones80, zeros128)
    dinvb, g0 = _dinv_g0(deg_p, x)

    h = x
    g = g0
    for li, (W, b) in enumerate(((W1, b1), (W2, b2), (W3, b3))):
        ci = W.shape[1]
        feat = ci == 256
        prop = _make_prop(feat)
        s1p = prop(g.reshape(-1, 128), src, dst, zeros128)
        out01, g1 = _mid(h, s1p, dinvb, W[0], W[1])
        s2p = prop(g1.reshape(-1, 128), src, dst, zeros128)
        last = li == 2
        res = _fin(out01, s2p, h, dinvb, W[2], b, last)
        if last:
            return res[0]
        h, g = res


# speedup vs baseline: 12.8150x; 1.0527x over previous
import functools

import jax
import jax.numpy as jnp
from jax import lax
from jax.experimental import pallas as pl
from jax.experimental.pallas import tpu as pltpu
from jax.experimental.pallas import tpu_sc as plsc

N = 10000
NP = 10240
E = 320000
NC = 2
NS = 16
F32 = jnp.float32


def _sc_mesh():
    return plsc.VectorSubcoreMesh(core_axis_name="c", subcore_axis_name="s")


@functools.cache
def _make_prop(feat_split):
    B = 80
    CPG = 2
    EPG = B * CPG
    W = 128
    NT = NS if feat_split else NC * NS
    EPT = E // NT
    G = EPT // EPG
    TAIL = (EPT % EPG) // B
    RPT = NP // NS

    scratch = (
        [pltpu.VMEM((EPG,), jnp.int32)] * 4 +
        [pltpu.VMEM((B,), jnp.int32)] * (4 * CPG) +
        [pltpu.VMEM((B, W), F32)] * (2 * CPG) +
        [pltpu.VMEM_SHARED((NP, W), F32)] +
        [pltpu.SemaphoreType.DMA] * 6
    )

    @functools.partial(
        pl.kernel,
        out_type=jax.ShapeDtypeStruct((NC, NP, W), F32),
        mesh=_sc_mesh(),
        scratch_types=scratch,
    )
    def prop_kernel(g2_hbm, src_hbm, dst_hbm, zeros_hbm, out_hbm, *sc):
        sblk = sc[0:2]
        dblk = sc[2:4]
        gidx = (sc[4:4 + CPG], sc[4 + CPG:4 + 2 * CPG])
        didx = (sc[4 + 2 * CPG:4 + 3 * CPG], sc[4 + 3 * CPG:4 + 4 * CPG])
        o = 4 + 4 * CPG
        rows = (sc[o:o + CPG], sc[o + CPG:o + 2 * CPG])
        acc = sc[o + 2 * CPG]
        gsem = sc[o + 2 * CPG + 1:o + 2 * CPG + 3]
        ssem = sc[o + 2 * CPG + 3:o + 2 * CPG + 5]
        asem = sc[o + 2 * CPG + 5:o + 2 * CPG + 7]

        c = lax.axis_index("c")
        s = lax.axis_index("s")
        e0 = (s if feat_split else s * NC + c) * EPT

        pltpu.sync_copy(zeros_hbm.at[pl.ds(s * RPT, RPT)],
                        acc.at[pl.ds(s * RPT, RPT)])
        plsc.subcore_barrier()

        def load_blk(bk, g, n=EPG):
            base = e0 + g * EPG
            pltpu.sync_copy(src_hbm.at[pl.ds(base, n)], sblk[bk])
            pltpu.sync_copy(dst_hbm.at[pl.ds(base, n)], dblk[bk])

        def fire_blk(bk, g):
            base = e0 + g * EPG
            pltpu.async_copy(src_hbm.at[pl.ds(base, EPG)], sblk[bk], asem[bk])
            pltpu.async_copy(dst_hbm.at[pl.ds(base, EPG)], dblk[bk], asem[bk])

        def wait_blk(bk):
            base = e0
            pltpu.make_async_copy(
                src_hbm.at[pl.ds(base, EPG)], sblk[bk], asem[bk]).wait()
            pltpu.make_async_copy(
                dst_hbm.at[pl.ds(base, EPG)], dblk[bk], asem[bk]).wait()

        def fire_gathers(bk, nch=CPG):
            for j in range(nch):
                for i in range(B // 16):
                    off = j * B + i * 16
                    s16 = sblk[bk][pl.ds(off, 16)]
                    if feat_split:
                        s16 = s16 * 2 + c
                    gidx[bk][j][pl.ds(i * 16, 16)] = s16
                    didx[bk][j][pl.ds(i * 16, 16)] = dblk[bk][pl.ds(off, 16)]
                pltpu.async_copy(g2_hbm.at[gidx[bk][j]], rows[bk][j], gsem[bk])

        def drain_gathers(bk, nch=CPG):
            for j in range(nch):
                pltpu.make_async_copy(
                    g2_hbm.at[gidx[bk][j]], rows[bk][j], gsem[bk]).wait()

        def fire_scatters(bk, nch=CPG):
            for j in range(nch):
                pltpu.async_copy(rows[bk][j], acc.at[didx[bk][j]], ssem[bk],
                                 add=True)

        def drain_scatters(bk, nch=CPG):
            for j in range(nch):
                pltpu.make_async_copy(
                    rows[bk][j], acc.at[didx[bk][j]], ssem[bk]).wait()

        load_blk(0, 0)
        fire_gathers(0)
        load_blk(1, 1)
        fire_gathers(1)

        def body(t, _):
            for bk in range(2):
                drain_gathers(bk)
                fire_scatters(bk)
                fire_blk(bk, 2 * t + 2 + bk)
                drain_scatters(bk)
                wait_blk(bk)
                fire_gathers(bk)
            return 0
        lax.fori_loop(0, (G - 2) // 2, body, 0)

        for bk in range(2):
            drain_gathers(bk)
            fire_scatters(bk)
            drain_scatters(bk)
        if G % 2:
            load_blk(0, G - 1)
            fire_gathers(0)
            drain_gathers(0)
            fire_scatters(0)
            drain_scatters(0)
        if TAIL:
            base = e0 + G * EPG
            pltpu.sync_copy(src_hbm.at[pl.ds(base, TAIL * B)],
                            sblk[0].at[pl.ds(0, TAIL * B)])
            pltpu.sync_copy(dst_hbm.at[pl.ds(base, TAIL * B)],
                            dblk[0].at[pl.ds(0, TAIL * B)])
            fire_gathers(0, TAIL)
            drain_gathers(0, TAIL)
            fire_scatters(0, TAIL)
            drain_scatters(0, TAIL)

        plsc.subcore_barrier()
        pltpu.sync_copy(acc.at[pl.ds(s * RPT, RPT)],
                        out_hbm.at[c, pl.ds(s * RPT, RPT)])

    return prop_kernel


@functools.cache
def _make_deg():
    B = 80
    CPG = 5
    EPG = B * CPG
    W = 128
    EPT = E // (NC * NS)
    G = EPT // EPG
    RPT = NP // NS

    scratch = (
        [pltpu.VMEM((EPG,), jnp.int32)] * 2 +
        [pltpu.VMEM((B,), jnp.int32)] * 10 +
        [pltpu.VMEM((B, W), F32)] +
        [pltpu.VMEM_SHARED((NP, W), F32)] +
        [pltpu.SemaphoreType.DMA] * 2
    )

    @functools.partial(
        pl.kernel,
        out_type=jax.ShapeDtypeStruct((NC, NP, W), F32),
        mesh=_sc_mesh(),
        scratch_types=scratch,
    )
    def deg_kernel(src_hbm, ones_hbm, zeros_hbm, out_hbm, *sc):
        sblk = sc[0:2]
        didx = (sc[2:7], sc[7:12])
        ones_v = sc[12]
        acc = sc[13]
        ssem = sc[14:16]

        c = lax.axis_index("c")
        s = lax.axis_index("s")
        e0 = (s * NC + c) * EPT

        pltpu.sync_copy(ones_hbm, ones_v)
        pltpu.sync_copy(zeros_hbm.at[pl.ds(s * RPT, RPT)],
                        acc.at[pl.ds(s * RPT, RPT)])
        plsc.subcore_barrier()

        def load_blk(bk, g):
            pltpu.sync_copy(src_hbm.at[pl.ds(e0 + g * EPG, EPG)], sblk[bk])

        def prep_fire(bk):
            for j in range(CPG):
                for i in range(B // 16):
                    didx[bk][j][pl.ds(i * 16, 16)] = \
                        sblk[bk][pl.ds(j * B + i * 16, 16)]
                pltpu.async_copy(ones_v, acc.at[didx[bk][j]], ssem[bk],
                                 add=True)

        def drain(bk):
            for j in range(CPG):
                pltpu.make_async_copy(
                    ones_v, acc.at[didx[bk][j]], ssem[bk]).wait()

        load_blk(0, 0)
        prep_fire(0)
        load_blk(1, 1)
        prep_fire(1)

        def body(t, _):
            for bk in range(2):
                drain(bk)
                load_blk(bk, 2 * t + 2 + bk)
                prep_fire(bk)
            return 0
        lax.fori_loop(0, (G - 2) // 2, body, 0)
        for bk in range(2):
            drain(bk)
        if G % 2:
            load_blk(0, G - 1)
            prep_fire(0)
            drain(0)

        plsc.subcore_barrier()
        pltpu.sync_copy(acc.at[pl.ds(s * RPT, RPT)],
                        out_hbm.at[c, pl.ds(s * RPT, RPT)])

    return deg_kernel


_R = 2000


def _tc_call(body, out_shapes, in_specs, out_specs):
    return pl.pallas_call(
        body,
        grid=(N // _R,),
        in_specs=in_specs,
        out_specs=out_specs,
        out_shape=out_shapes,
    )


def _rows(c):
    return pl.BlockSpec((_R, c), lambda i: (i, 0))


def _pair(w):
    return pl.BlockSpec((NC, _R, w), lambda i: (0, i, 0))


def _full(a, b):
    return pl.BlockSpec((a, b), lambda i: (0, 0))


def _dinv_g0(deg_p, x):
    def body(degp_ref, x_ref, dinvb_ref, g0_ref):
        deg = degp_ref[0][:, 0:1] + degp_ref[1][:, 0:1]
        dinv = jnp.where(deg > 0.0, lax.rsqrt(deg), 0.0)
        dinvb_ref[...] = jnp.broadcast_to(dinv, (_R, 128))
        g0_ref[...] = dinv * x_ref[...]

    return _tc_call(
        body,
        (jax.ShapeDtypeStruct((N, 128), F32), jax.ShapeDtypeStruct((N, 128), F32)),
        [_pair(128), _rows(128)],
        (_rows(128), _rows(128)),
    )(deg_p, x)


def _pre(h, w0, w2):
    ci, co = w0.shape

    def body(h_ref, w0_ref, w2_ref, p01_ref, p2_ref):
        p01_ref[...] = jnp.dot(h_ref[...], w0_ref[...],
                               preferred_element_type=F32)
        p2_ref[...] = jnp.dot(h_ref[...], w2_ref[...],
                              preferred_element_type=F32)

    return _tc_call(
        body,
        (jax.ShapeDtypeStruct((N, co), F32), jax.ShapeDtypeStruct((N, co), F32)),
        [_rows(ci), _full(ci, co), _full(ci, co)],
        (_rows(co), _rows(co)),
    )(h, w0, w2)


def _mid(pre01, s1p, dinvb, w1):
    ci, co = w1.shape

    def body(p_ref, sp_ref, dv_ref, w1_ref, out01_ref, g1_ref):
        dv = dv_ref[:, 0:1]
        if ci == 256:
            s1 = jnp.concatenate([sp_ref[0], sp_ref[1]], axis=-1)
        else:
            s1 = sp_ref[0] + sp_ref[1]
        tx1 = (-dv) * s1
        out01_ref[...] = p_ref[...] + jnp.dot(
            tx1, w1_ref[...], preferred_element_type=F32)
        g1_ref[...] = dv * tx1

    return _tc_call(
        body,
        (jax.ShapeDtypeStruct((N, co), F32), jax.ShapeDtypeStruct((N, ci), F32)),
        [_rows(co), _pair(128), _rows(128), _full(ci, co)],
        (_rows(co), _rows(ci)),
    )(pre01, s1p, dinvb, w1)


def _fin(out01, s2p, pre2, dinvb, w2, b, last):
    ci, co = w2.shape

    def body(o_ref, sp_ref, p2_ref, dv_ref, w2_ref, b_ref, *outs):
        dv = dv_ref[:, 0:1]
        if ci == 256:
            s2 = jnp.concatenate([sp_ref[0], sp_ref[1]], axis=-1)
        else:
            s2 = sp_ref[0] + sp_ref[1]
        tx2 = (-2.0 * dv) * s2
        out = (o_ref[...] - p2_ref[...] + b_ref[...]
               + jnp.dot(tx2, w2_ref[...], preferred_element_type=F32))
        a = jnp.maximum(out, 0.0)
        if last:
            m = jnp.max(a, axis=-1, keepdims=True)
            lse = m + jnp.log(jnp.sum(jnp.exp(a - m), axis=-1, keepdims=True))
            outs[0][...] = a - lse
        else:
            outs[0][...] = a
            outs[1][...] = dv * a

    n_out = 1 if last else 2
    return _tc_call(
        body,
        tuple(jax.ShapeDtypeStruct((N, co), F32) for _ in range(n_out)),
        [_rows(co), _pair(128), _rows(co), _rows(128), _full(ci, co),
         pl.BlockSpec((1, co), lambda i: (0, 0))],
        tuple(_rows(co) for _ in range(n_out)),
    )(out01, s2p, pre2, dinvb, w2, b.reshape(1, co))


@jax.jit
def kernel(x, edge_index, edge_attr, W1, b1, W2, b2, W3, b3):
    src = edge_index[0]
    dst = edge_index[1]
    zeros128 = jnp.zeros((NP, 128), F32)

    ones80 = jnp.ones((80, 128), F32)
    deg_p = _make_deg()(src, ones80, zeros128)
    dinvb, g0 = _dinv_g0(deg_p, x)

    h = x
    g = g0
    for li, (W, b) in enumerate(((W1, b1), (W2, b2), (W3, b3))):
        ci = W.shape[1]
        feat = ci == 256
        prop = _make_prop(feat)
        pre01, pre2 = _pre(h, W[0], W[2])
        s1p = prop(g.reshape(-1, 128), src, dst, zeros128)
        out01, g1 = _mid(pre01, s1p, dinvb, W[1])
        s2p = prop(g1.reshape(-1, 128), src, dst, zeros128)
        last = li == 2
        res = _fin(out01, s2p, pre2, dinvb, W[2], b, last)
        if last:
            return res[0]
        h, g = res
